# Initial kernel scaffold; baseline (speedup 1.0000x reference)
import functools

import jax
import jax.numpy as jnp
from jax import lax
from jax.experimental import pallas as pl
from jax.experimental.pallas import tpu as pltpu
from jax.experimental.pallas import tpu_sc as plsc

N = 10000
E = 320000
D = 128
ACC_W = 144

NC = 2
NS = 16
NW = NC * NS
EPW = E // NW
SB = 80
NSTR = 5
C = SB * NSTR
NCHUNK = EPW // C
ROWS_PT = N // NS

BM = 1000
GRID = N // BM


def _mm_first_body(x_ref, w_ref, a8_ref, z_ref, st_ref):
    xb = x_ref[...]
    w = w_ref[...]
    z_ref[...] = jnp.dot(xb, w, preferred_element_type=jnp.float32)
    a8 = a8_ref[...]
    A8 = lax.dot_general(a8, w, (((1,), (1,)), ((), ())),
                         preferred_element_type=jnp.float32)
    st_ref[...] = lax.dot_general(A8, xb, (((1,), (1,)), ((), ())),
                                  preferred_element_type=jnp.float32)


def _mm_next_body(acc_ref, w_ref, a8_ref, z_ref, st_ref):
    acc = acc_ref[...]
    u = acc[0, :, :D] + acc[1, :, :D]
    den = acc[0, :, D:D + 1] + acc[1, :, D:D + 1]
    h = jnp.maximum(u / (den + 1e-16), 0.0)
    w = w_ref[...]
    z_ref[...] = jnp.dot(h, w, preferred_element_type=jnp.float32)
    a8 = a8_ref[...]
    A8 = lax.dot_general(a8, w, (((1,), (1,)), ((), ())),
                         preferred_element_type=jnp.float32)
    st_ref[...] = lax.dot_general(A8, h, (((1,), (1,)), ((), ())),
                                  preferred_element_type=jnp.float32)


def _final_body(acc_ref, o_ref):
    acc = acc_ref[...]
    u = acc[0, :, :D] + acc[1, :, :D]
    den = acc[0, :, D:D + 1] + acc[1, :, D:D + 1]
    o_ref[...] = u / (den + 1e-16)


_mm_first = pl.pallas_call(
    _mm_first_body,
    grid=(GRID,),
    in_specs=[
        pl.BlockSpec((BM, D), lambda i: (i, 0)),
        pl.BlockSpec((D, D), lambda i: (0, 0)),
        pl.BlockSpec((8, D), lambda i: (0, 0)),
    ],
    out_specs=[
        pl.BlockSpec((BM, D), lambda i: (i, 0)),
        pl.BlockSpec((8, BM), lambda i: (0, i)),
    ],
    out_shape=[
        jax.ShapeDtypeStruct((N, D), jnp.float32),
        jax.ShapeDtypeStruct((8, N), jnp.float32),
    ],
)

_mm_next = pl.pallas_call(
    _mm_next_body,
    grid=(GRID,),
    in_specs=[
        pl.BlockSpec((NC, BM, ACC_W), lambda i: (0, i, 0)),
        pl.BlockSpec((D, D), lambda i: (0, 0)),
        pl.BlockSpec((8, D), lambda i: (0, 0)),
    ],
    out_specs=[
        pl.BlockSpec((BM, D), lambda i: (i, 0)),
        pl.BlockSpec((8, BM), lambda i: (0, i)),
    ],
    out_shape=[
        jax.ShapeDtypeStruct((N, D), jnp.float32),
        jax.ShapeDtypeStruct((8, N), jnp.float32),
    ],
)

_final = pl.pallas_call(
    _final_body,
    grid=(GRID,),
    in_specs=[pl.BlockSpec((NC, BM, ACC_W), lambda i: (0, i, 0))],
    out_specs=pl.BlockSpec((BM, D), lambda i: (i, 0)),
    out_shape=jax.ShapeDtypeStruct((N, D), jnp.float32),
)


def _sc_edge_body(z_hbm, st_hbm, ei_hbm, out_hbm,
                  acc, s_v, t_v, src_v, dst_v, ex_v, rows_v, stage_v,
                  gsem, ssem):
    cid = lax.axis_index("c")
    sid = lax.axis_index("s")
    wid = sid * NC + cid

    pltpu.sync_copy(st_hbm.at[0], s_v)
    pltpu.sync_copy(st_hbm.at[1], t_v)

    zeros16 = jnp.zeros((16,), jnp.float32)

    @pl.loop(0, C)
    def _zero_stage(i):
        for k in range(ACC_W // 16):
            stage_v[i, pl.ds(k * 16, 16)] = zeros16

    base_row = sid * ROWS_PT
    pltpu.sync_copy(stage_v, acc.at[pl.ds(base_row, C)])
    pltpu.sync_copy(stage_v.at[pl.ds(0, ROWS_PT - C)],
                    acc.at[pl.ds(base_row + C, ROWS_PT - C)])
    plsc.subcore_barrier()

    ebase = wid * EPW

    @pl.loop(0, NCHUNK)
    def _chunk(ci):
        off = ebase + ci * C
        for j in range(NSTR):
            pltpu.sync_copy(ei_hbm.at[0, pl.ds(off + j * SB, SB)], src_v.at[j])
            pltpu.sync_copy(ei_hbm.at[1, pl.ds(off + j * SB, SB)], dst_v.at[j])

        for j in range(NSTR):
            for k in range(SB // 16):
                si = src_v[j, pl.ds(k * 16, 16)]
                di = dst_v[j, pl.ds(k * 16, 16)]
                e = plsc.load_gather(s_v, [si]) + plsc.load_gather(t_v, [di])
                e = jnp.where(e < 0.0, e * 0.2, e)
                ex_v[pl.ds(j * SB + k * 16, 16)] = jnp.exp(e)

        cps = [pltpu.async_copy(z_hbm.at[src_v.at[j]],
                                rows_v.at[pl.ds(j * SB, SB)], gsem)
               for j in range(NSTR)]
        for cp in cps:
            cp.wait()

        @pl.loop(0, C)
        def _scale(i):
            exb = plsc.load_gather(ex_v, [jnp.full((16,), i, jnp.int32)])
            for k in range(D // 16):
                stage_v[i, pl.ds(k * 16, 16)] = rows_v[i, pl.ds(k * 16, 16)] * exb
            stage_v[i, pl.ds(D, 16)] = exb

        for j in range(NSTR):
            pltpu.sync_copy(stage_v.at[pl.ds(j * SB, SB)],
                            acc.at[dst_v.at[j]], add=True)

    plsc.subcore_barrier()
    pltpu.sync_copy(acc.at[pl.ds(base_row, ROWS_PT)],
                    out_hbm.at[cid, pl.ds(base_row, ROWS_PT)])


_sc_edge = functools.partial(
    pl.kernel,
    out_type=jax.ShapeDtypeStruct((NC, N, ACC_W), jnp.float32),
    mesh=plsc.VectorSubcoreMesh(core_axis_name="c", subcore_axis_name="s"),
    scratch_types=[
        pltpu.VMEM_SHARED((N, ACC_W), jnp.float32),
        pltpu.VMEM((N,), jnp.float32),
        pltpu.VMEM((N,), jnp.float32),
        pltpu.VMEM((NSTR, SB), jnp.int32),
        pltpu.VMEM((NSTR, SB), jnp.int32),
        pltpu.VMEM((C,), jnp.float32),
        pltpu.VMEM((C, D), jnp.float32),
        pltpu.VMEM((C, ACC_W), jnp.float32),
        pltpu.SemaphoreType.DMA,
        pltpu.SemaphoreType.DMA,
    ],
)(_sc_edge_body)


def _a8(a):
    return jnp.zeros((8, D), jnp.float32).at[0].set(a[:D]).at[1].set(a[D:])


@jax.jit
def kernel(x, edge_index, W1, a1, W2, a2, W3, a3):
    z1, st1 = _mm_first(x, W1, _a8(a1))
    acc1 = _sc_edge(z1, st1, edge_index)
    z2, st2 = _mm_next(acc1, W2, _a8(a2))
    acc2 = _sc_edge(z2, st2, edge_index)
    z3, st3 = _mm_next(acc2, W3, _a8(a3))
    acc3 = _sc_edge(z3, st3, edge_index)
    return _final(acc3)


# speedup vs baseline: 9.6117x; 9.6117x over previous
import functools

import jax
import jax.numpy as jnp
from jax import lax
from jax.experimental import pallas as pl
from jax.experimental.pallas import tpu as pltpu
from jax.experimental.pallas import tpu_sc as plsc

N = 10000
E = 320000
D = 128
ACC_W = 144

NC = 2
NS = 16
NW = NC * NS
EPW = E // NW
ROWS_PT = N // NS

CA = 2000
NCA = EPW // CA
CB = 80
NCB = EPW // CB

_SC_PARAMS = pltpu.CompilerParams(use_tc_tiling_on_sc=False,
                                  needs_layout_passes=False)
_SC_MESH = plsc.VectorSubcoreMesh(core_axis_name="c", subcore_axis_name="s",
                                  num_cores=NC, num_subcores=NS)


def _mm_first_body(x_ref, w_ref, a8_ref, z_ref, st_ref):
    xb = x_ref[...]
    w = w_ref[...]
    z_ref[...] = jnp.dot(xb, w, preferred_element_type=jnp.float32)
    a8 = a8_ref[...]
    A8 = lax.dot_general(a8, w, (((1,), (1,)), ((), ())),
                         preferred_element_type=jnp.float32)
    st_ref[...] = lax.dot_general(A8, xb, (((1,), (1,)), ((), ())),
                                  preferred_element_type=jnp.float32)


def _mm_next_body(acc_ref, w_ref, a8_ref, z_ref, st_ref):
    acc = acc_ref[...]
    u = acc[0, :, :D] + acc[1, :, :D]
    den = acc[0, :, D:D + 1] + acc[1, :, D:D + 1]
    h = jnp.maximum(u / (den + 1e-16), 0.0)
    w = w_ref[...]
    z_ref[...] = jnp.dot(h, w, preferred_element_type=jnp.float32)
    a8 = a8_ref[...]
    A8 = lax.dot_general(a8, w, (((1,), (1,)), ((), ())),
                         preferred_element_type=jnp.float32)
    st_ref[...] = lax.dot_general(A8, h, (((1,), (1,)), ((), ())),
                                  preferred_element_type=jnp.float32)


def _final_body(acc_ref, o_ref):
    acc = acc_ref[...]
    u = acc[0, :, :D] + acc[1, :, :D]
    den = acc[0, :, D:D + 1] + acc[1, :, D:D + 1]
    o_ref[...] = u / (den + 1e-16)


_mm_first = pl.pallas_call(
    _mm_first_body,
    out_shape=[
        jax.ShapeDtypeStruct((N, D), jnp.float32),
        jax.ShapeDtypeStruct((8, N), jnp.float32),
    ],
)

_mm_next = pl.pallas_call(
    _mm_next_body,
    out_shape=[
        jax.ShapeDtypeStruct((N, D), jnp.float32),
        jax.ShapeDtypeStruct((8, N), jnp.float32),
    ],
)

_final = pl.pallas_call(
    _final_body,
    out_shape=jax.ShapeDtypeStruct((N, D), jnp.float32),
)


def _sc_scal_body(st_hbm, ei_hbm, ex_hbm, s_v, t_v, src_v, dst_v, exv_v):
    cid = lax.axis_index("c")
    sid = lax.axis_index("s")
    wid = sid * NC + cid

    pltpu.sync_copy(st_hbm.at[0], s_v)
    pltpu.sync_copy(st_hbm.at[1], t_v)

    ebase = wid * EPW

    @pl.loop(0, NCA)
    def _chunk(ci):
        off = ebase + ci * CA
        pltpu.sync_copy(ei_hbm.at[0, pl.ds(off, CA)], src_v)
        pltpu.sync_copy(ei_hbm.at[1, pl.ds(off, CA)], dst_v)

        @pl.loop(0, CA // 16)
        def _vec(k):
            si = src_v[pl.ds(k * 16, 16)]
            di = dst_v[pl.ds(k * 16, 16)]
            e = plsc.load_gather(s_v, [si]) + plsc.load_gather(t_v, [di])
            e = jnp.where(e < 0.0, e * 0.2, e)
            exv_v[pl.ds(k * 16, 16)] = jnp.exp(e)

        pltpu.sync_copy(exv_v, ex_hbm.at[pl.ds(off, CA)])


_sc_scal = functools.partial(
    pl.kernel,
    out_type=jax.ShapeDtypeStruct((E,), jnp.float32),
    mesh=_SC_MESH,
    scratch_types=[
        pltpu.VMEM((N,), jnp.float32),
        pltpu.VMEM((N,), jnp.float32),
        pltpu.VMEM((CA,), jnp.int32),
        pltpu.VMEM((CA,), jnp.int32),
        pltpu.VMEM((CA,), jnp.float32),
    ],
    compiler_params=_SC_PARAMS,
)(_sc_scal_body)


def _sc_row_body(z_hbm, ei_hbm, ex_hbm, out_hbm,
                 acc, src_v, dst_v, ex_v, rows_v, stage_v):
    cid = lax.axis_index("c")
    sid = lax.axis_index("s")
    wid = sid * NC + cid

    zeros16 = jnp.zeros((16,), jnp.float32)

    @pl.loop(0, CB)
    def _zero_stage(i):
        for k in range(ACC_W // 16):
            stage_v[i, pl.ds(k * 16, 16)] = zeros16

    base_row = sid * ROWS_PT
    for r in range(7):
        pltpu.sync_copy(stage_v, acc.at[pl.ds(base_row + r * CB, CB)])
    pltpu.sync_copy(stage_v.at[pl.ds(0, ROWS_PT - 7 * CB)],
                    acc.at[pl.ds(base_row + 7 * CB, ROWS_PT - 7 * CB)])
    plsc.subcore_barrier()

    ebase = wid * EPW

    @pl.loop(0, NCB)
    def _chunk(ci):
        off = ebase + ci * CB
        pltpu.sync_copy(ei_hbm.at[0, pl.ds(off, CB)], src_v)
        pltpu.sync_copy(ei_hbm.at[1, pl.ds(off, CB)], dst_v)
        pltpu.sync_copy(ex_hbm.at[pl.ds(off, CB)], ex_v)
        pltpu.sync_copy(z_hbm.at[src_v], rows_v)

        @pl.loop(0, CB)
        def _scale(i):
            exb = plsc.load_gather(ex_v, [jnp.full((16,), i, jnp.int32)])
            for k in range(D // 16):
                stage_v[i, pl.ds(k * 16, 16)] = rows_v[i, pl.ds(k * 16, 16)] * exb
            stage_v[i, pl.ds(D, 16)] = exb

        pltpu.sync_copy(stage_v, acc.at[dst_v], add=True)

    plsc.subcore_barrier()
    pltpu.sync_copy(acc.at[pl.ds(base_row, ROWS_PT)],
                    out_hbm.at[cid, pl.ds(base_row, ROWS_PT)])


_sc_row = functools.partial(
    pl.kernel,
    out_type=jax.ShapeDtypeStruct((NC, N, ACC_W), jnp.float32),
    mesh=_SC_MESH,
    scratch_types=[
        pltpu.VMEM_SHARED((N, ACC_W), jnp.float32),
        pltpu.VMEM((CB,), jnp.int32),
        pltpu.VMEM((CB,), jnp.int32),
        pltpu.VMEM((CB,), jnp.float32),
        pltpu.VMEM((CB, D), jnp.float32),
        pltpu.VMEM((CB, ACC_W), jnp.float32),
    ],
    compiler_params=_SC_PARAMS,
)(_sc_row_body)


def _gat_sc(z, st, edge_index):
    ex = _sc_scal(st, edge_index)
    return _sc_row(z, edge_index, ex)


def _a8(a):
    return jnp.zeros((8, D), jnp.float32).at[0].set(a[:D]).at[1].set(a[D:])


@jax.jit
def kernel(x, edge_index, W1, a1, W2, a2, W3, a3):
    z1, st1 = _mm_first(x, W1, _a8(a1))
    acc1 = _gat_sc(z1, st1, edge_index)
    z2, st2 = _mm_next(acc1, W2, _a8(a2))
    acc2 = _gat_sc(z2, st2, edge_index)
    z3, st3 = _mm_next(acc2, W3, _a8(a3))
    acc3 = _gat_sc(z3, st3, edge_index)
    return _final(acc3)


# speedup vs baseline: 25.7479x; 2.6788x over previous
import functools

import jax
import jax.numpy as jnp
from jax import lax
from jax.experimental import pallas as pl
from jax.experimental.pallas import tpu as pltpu
from jax.experimental.pallas import tpu_sc as plsc

N = 10000
E = 320000
D = 128
DW = 16

NC = 2
NS = 16
NW = NC * NS
EPW = E // NW
ROWS_PT = N // NS

CB = 80
CA = 2000
KPS = CA // CB
NSUP = EPW // CA

_SC_PARAMS = pltpu.CompilerParams(use_tc_tiling_on_sc=False,
                                  needs_layout_passes=False)
_SC_MESH = plsc.VectorSubcoreMesh(core_axis_name="c", subcore_axis_name="s",
                                  num_cores=NC, num_subcores=NS)


def _mm_first_body(x_ref, w_ref, a8_ref, z_ref, st_ref):
    xb = x_ref[...]
    w = w_ref[...]
    z_ref[...] = jnp.dot(xb, w, preferred_element_type=jnp.float32)
    a8 = a8_ref[...]
    A8 = lax.dot_general(a8, w, (((1,), (1,)), ((), ())),
                         preferred_element_type=jnp.float32)
    st_ref[...] = lax.dot_general(A8, xb, (((1,), (1,)), ((), ())),
                                  preferred_element_type=jnp.float32)


def _mm_next_body(acc_ref, accd_ref, w_ref, a8_ref, z_ref, st_ref):
    u = acc_ref[0] + acc_ref[1]
    den = accd_ref[0, :, 0:1] + accd_ref[1, :, 0:1]
    h = jnp.maximum(u / (den + 1e-16), 0.0)
    w = w_ref[...]
    z_ref[...] = jnp.dot(h, w, preferred_element_type=jnp.float32)
    a8 = a8_ref[...]
    A8 = lax.dot_general(a8, w, (((1,), (1,)), ((), ())),
                         preferred_element_type=jnp.float32)
    st_ref[...] = lax.dot_general(A8, h, (((1,), (1,)), ((), ())),
                                  preferred_element_type=jnp.float32)


def _final_body(acc_ref, accd_ref, o_ref):
    u = acc_ref[0] + acc_ref[1]
    den = accd_ref[0, :, 0:1] + accd_ref[1, :, 0:1]
    o_ref[...] = u / (den + 1e-16)


_mm_first = pl.pallas_call(
    _mm_first_body,
    out_shape=[
        jax.ShapeDtypeStruct((N, D), jnp.float32),
        jax.ShapeDtypeStruct((8, N), jnp.float32),
    ],
)

_mm_next = pl.pallas_call(
    _mm_next_body,
    out_shape=[
        jax.ShapeDtypeStruct((N, D), jnp.float32),
        jax.ShapeDtypeStruct((8, N), jnp.float32),
    ],
)

_final = pl.pallas_call(
    _final_body,
    out_shape=jax.ShapeDtypeStruct((N, D), jnp.float32),
)


def _sc_scal_body(st_hbm, ei_hbm, dst3_hbm, ex_hbm, outd_hbm,
                  s_v, t_v, src_v, dst_v, exv_v, staged_v, accd, dsem):
    cid = lax.axis_index("c")
    sid = lax.axis_index("s")
    wid = sid * NC + cid

    pltpu.sync_copy(st_hbm.at[0], s_v)
    pltpu.sync_copy(st_hbm.at[1], t_v)

    zeros16 = jnp.zeros((16,), jnp.float32)

    @pl.loop(0, CA)
    def _zero_staged(i):
        staged_v[i, pl.ds(0, DW)] = zeros16

    base_row = sid * ROWS_PT
    pltpu.sync_copy(staged_v.at[pl.ds(0, ROWS_PT)],
                    accd.at[pl.ds(base_row, ROWS_PT)])
    plsc.subcore_barrier()

    cbase = wid * (EPW // CB)

    @pl.loop(0, NSUP)
    def _super(ci):
        soff = cbase + ci * KPS
        eoff = soff * CB
        pltpu.sync_copy(ei_hbm.at[0, pl.ds(eoff, CA)], src_v)
        pltpu.sync_copy(dst3_hbm.at[pl.ds(soff, KPS)], dst_v)

        @pl.loop(0, KPS)
        def _row(r):
            for s in range(CB // 16):
                si = src_v[pl.ds(r * CB + s * 16, 16)]
                di = dst_v[r, pl.ds(s * 16, 16)]
                e = plsc.load_gather(s_v, [si]) + plsc.load_gather(t_v, [di])
                e = jnp.where(e < 0.0, e * 0.2, e)
                exv_v[pl.ds(r * CB + s * 16, 16)] = jnp.exp(e)

        pltpu.sync_copy(exv_v, ex_hbm.at[pl.ds(eoff, CA)])

        @pl.loop(0, CA)
        def _build(i):
            staged_v[i, pl.ds(0, DW)] = plsc.load_gather(
                exv_v, [jnp.full((16,), i, jnp.int32)])

        cps = [pltpu.async_copy(staged_v.at[pl.ds(k * CB, CB)],
                                accd.at[dst_v.at[k]], dsem, add=True)
               for k in range(KPS)]
        for cp in cps:
            cp.wait()

    plsc.subcore_barrier()
    pltpu.sync_copy(accd.at[pl.ds(base_row, ROWS_PT)],
                    outd_hbm.at[cid, pl.ds(base_row, ROWS_PT)])


_sc_scal = functools.partial(
    pl.kernel,
    out_type=[
        jax.ShapeDtypeStruct((E,), jnp.float32),
        jax.ShapeDtypeStruct((NC, N, DW), jnp.float32),
    ],
    mesh=_SC_MESH,
    scratch_types=[
        pltpu.VMEM((N,), jnp.float32),
        pltpu.VMEM((N,), jnp.float32),
        pltpu.VMEM((CA,), jnp.int32),
        pltpu.VMEM((KPS, CB), jnp.int32),
        pltpu.VMEM((CA,), jnp.float32),
        pltpu.VMEM((CA, DW), jnp.float32),
        pltpu.VMEM_SHARED((N, DW), jnp.float32),
        pltpu.SemaphoreType.DMA,
    ],
    compiler_params=_SC_PARAMS,
)(_sc_scal_body)


def _sc_row_body(z_hbm, src_hbm, dst3_hbm, ex_hbm, out_hbm,
                 acc, src_v, dst_v, ex_v, r0, r1, r2,
                 g0, g1, g2, s0, s1, s2):
    cid = lax.axis_index("c")
    sid = lax.axis_index("s")
    wid = sid * NC + cid
    rows = [r0, r1, r2]
    gsem = [g0, g1, g2]
    ssem = [s0, s1, s2]

    zeros16 = jnp.zeros((16,), jnp.float32)

    @pl.loop(0, CB)
    def _zero_r0(i):
        for c in range(D // 16):
            r0[i, pl.ds(c * 16, 16)] = zeros16

    base_row = sid * ROWS_PT
    for r in range(7):
        pltpu.sync_copy(r0, acc.at[pl.ds(base_row + r * CB, CB)])
    pltpu.sync_copy(r0.at[pl.ds(0, ROWS_PT - 7 * CB)],
                    acc.at[pl.ds(base_row + 7 * CB, ROWS_PT - 7 * CB)])
    plsc.subcore_barrier()

    cbase = wid * (EPW // CB)

    @pl.loop(0, NSUP)
    def _super(ci):
        soff = cbase + ci * KPS
        eoff = soff * CB
        pltpu.sync_copy(src_hbm.at[pl.ds(eoff, CA)], src_v)
        pltpu.sync_copy(dst3_hbm.at[pl.ds(soff, KPS)], dst_v)
        pltpu.sync_copy(ex_hbm.at[pl.ds(eoff, CA)], ex_v)

        def gather(k):
            b = k % 3
            return pltpu.async_copy(
                z_hbm.at[src_v.at[pl.ds(k * CB, CB)]], rows[b], gsem[b])

        def scale(k):
            buf = rows[k % 3]

            @pl.loop(0, CB)
            def _scale(i):
                exb = plsc.load_gather(
                    ex_v, [jnp.full((16,), k * CB, jnp.int32) + i])
                for c in range(D // 16):
                    buf[i, pl.ds(c * 16, 16)] = buf[i, pl.ds(c * 16, 16)] * exb

        def scatter(k):
            b = k % 3
            return pltpu.async_copy(rows[b], acc.at[dst_v.at[k]],
                                    ssem[b], add=True)

        gcps = {0: gather(0), 1: gather(1)}
        scps = {}
        for k in range(KPS):
            if k + 2 <= KPS - 1:
                if k >= 1:
                    scps[k - 1].wait()
                gcps[k + 2] = gather(k + 2)
            gcps[k].wait()
            scale(k)
            scps[k] = scatter(k)
        for k in (KPS - 3, KPS - 2, KPS - 1):
            scps[k].wait()

    plsc.subcore_barrier()
    pltpu.sync_copy(acc.at[pl.ds(base_row, ROWS_PT)],
                    out_hbm.at[cid, pl.ds(base_row, ROWS_PT)])


_sc_row = functools.partial(
    pl.kernel,
    out_type=jax.ShapeDtypeStruct((NC, N, D), jnp.float32),
    mesh=_SC_MESH,
    scratch_types=[
        pltpu.VMEM_SHARED((N, D), jnp.float32),
        pltpu.VMEM((CA,), jnp.int32),
        pltpu.VMEM((KPS, CB), jnp.int32),
        pltpu.VMEM((CA,), jnp.float32),
        pltpu.VMEM((CB, D), jnp.float32),
        pltpu.VMEM((CB, D), jnp.float32),
        pltpu.VMEM((CB, D), jnp.float32),
        pltpu.SemaphoreType.DMA,
        pltpu.SemaphoreType.DMA,
        pltpu.SemaphoreType.DMA,
        pltpu.SemaphoreType.DMA,
        pltpu.SemaphoreType.DMA,
        pltpu.SemaphoreType.DMA,
    ],
    compiler_params=_SC_PARAMS,
)(_sc_row_body)


def _gat_sc(z, st, edge_index, src_flat, dst3):
    ex, accd = _sc_scal(st, edge_index, dst3)
    acc = _sc_row(z, src_flat, dst3, ex)
    return acc, accd


def _a8(a):
    return jnp.zeros((8, D), jnp.float32).at[0].set(a[:D]).at[1].set(a[D:])


@jax.jit
def kernel(x, edge_index, W1, a1, W2, a2, W3, a3):
    src_flat = edge_index[0]
    dst3 = edge_index[1].reshape(E // CB, CB)
    z1, st1 = _mm_first(x, W1, _a8(a1))
    acc1, accd1 = _gat_sc(z1, st1, edge_index, src_flat, dst3)
    z2, st2 = _mm_next(acc1, accd1, W2, _a8(a2))
    acc2, accd2 = _gat_sc(z2, st2, edge_index, src_flat, dst3)
    z3, st3 = _mm_next(acc2, accd2, W3, _a8(a3))
    acc3, accd3 = _gat_sc(z3, st3, edge_index, src_flat, dst3)
    return _final(acc3, accd3)


# speedup vs baseline: 29.3304x; 1.1391x over previous
import functools

import jax
import jax.numpy as jnp
from jax import lax
from jax.experimental import pallas as pl
from jax.experimental.pallas import tpu as pltpu
from jax.experimental.pallas import tpu_sc as plsc

N = 10000
E = 320000
D = 128
ZW = 144

NC = 2
NS = 16
NW = NC * NS
EPW = E // NW
ROWS_PT = N // NS

CB = 80
CA = 2000
KPS = CA // CB
NSUP = EPW // CA

_SC_PARAMS = pltpu.CompilerParams(use_tc_tiling_on_sc=False,
                                  needs_layout_passes=False)
_SC_MESH = plsc.VectorSubcoreMesh(core_axis_name="c", subcore_axis_name="s",
                                  num_cores=NC, num_subcores=NS)


def _zpad(z):
    ones = jnp.ones((N, 1), jnp.float32)
    zeros = jnp.zeros((N, ZW - D - 1), jnp.float32)
    return jnp.concatenate([z, ones, zeros], axis=1)


def _mm_first_body(x_ref, w_ref, a8_ref, z_ref, st_ref):
    xb = x_ref[...]
    w = w_ref[...]
    z = jnp.dot(xb, w, preferred_element_type=jnp.float32)
    z_ref[...] = _zpad(z)
    a8 = a8_ref[...]
    A8 = lax.dot_general(a8, w, (((1,), (1,)), ((), ())),
                         preferred_element_type=jnp.float32)
    st_ref[...] = lax.dot_general(A8, xb, (((1,), (1,)), ((), ())),
                                  preferred_element_type=jnp.float32)


def _mm_next_body(acc_ref, w_ref, a8_ref, z_ref, st_ref):
    acc = acc_ref[...]
    u = acc[0, :, :D] + acc[1, :, :D]
    den = acc[0, :, D:D + 1] + acc[1, :, D:D + 1]
    h = jnp.maximum(u / (den + 1e-16), 0.0)
    w = w_ref[...]
    z = jnp.dot(h, w, preferred_element_type=jnp.float32)
    z_ref[...] = _zpad(z)
    a8 = a8_ref[...]
    A8 = lax.dot_general(a8, w, (((1,), (1,)), ((), ())),
                         preferred_element_type=jnp.float32)
    st_ref[...] = lax.dot_general(A8, h, (((1,), (1,)), ((), ())),
                                  preferred_element_type=jnp.float32)


def _final_body(acc_ref, o_ref):
    acc = acc_ref[...]
    u = acc[0, :, :D] + acc[1, :, :D]
    den = acc[0, :, D:D + 1] + acc[1, :, D:D + 1]
    o_ref[...] = u / (den + 1e-16)


_mm_first = pl.pallas_call(
    _mm_first_body,
    out_shape=[
        jax.ShapeDtypeStruct((N, ZW), jnp.float32),
        jax.ShapeDtypeStruct((8, N), jnp.float32),
    ],
)

_mm_next = pl.pallas_call(
    _mm_next_body,
    out_shape=[
        jax.ShapeDtypeStruct((N, ZW), jnp.float32),
        jax.ShapeDtypeStruct((8, N), jnp.float32),
    ],
)

_final = pl.pallas_call(
    _final_body,
    out_shape=jax.ShapeDtypeStruct((N, D), jnp.float32),
)


def _sc_scal_body(st_hbm, ei_hbm, ex_hbm, s_v, t_v, src_v, dst_v, exv_v):
    cid = lax.axis_index("c")
    sid = lax.axis_index("s")
    wid = sid * NC + cid

    pltpu.sync_copy(st_hbm.at[0], s_v)
    pltpu.sync_copy(st_hbm.at[1], t_v)

    ebase = wid * EPW

    @pl.loop(0, NSUP)
    def _chunk(ci):
        off = ebase + ci * CA
        pltpu.sync_copy(ei_hbm.at[0, pl.ds(off, CA)], src_v)
        pltpu.sync_copy(ei_hbm.at[1, pl.ds(off, CA)], dst_v)

        @pl.loop(0, CA // 16)
        def _vec(k):
            si = src_v[pl.ds(k * 16, 16)]
            di = dst_v[pl.ds(k * 16, 16)]
            e = plsc.load_gather(s_v, [si]) + plsc.load_gather(t_v, [di])
            e = jnp.where(e < 0.0, e * 0.2, e)
            exv_v[pl.ds(k * 16, 16)] = jnp.exp(e)

        pltpu.sync_copy(exv_v, ex_hbm.at[pl.ds(off, CA)])


_sc_scal = functools.partial(
    pl.kernel,
    out_type=jax.ShapeDtypeStruct((E,), jnp.float32),
    mesh=_SC_MESH,
    scratch_types=[
        pltpu.VMEM((N,), jnp.float32),
        pltpu.VMEM((N,), jnp.float32),
        pltpu.VMEM((CA,), jnp.int32),
        pltpu.VMEM((CA,), jnp.int32),
        pltpu.VMEM((CA,), jnp.float32),
    ],
    compiler_params=_SC_PARAMS,
)(_sc_scal_body)


def _sc_row_body(z_hbm, src_hbm, dst3_hbm, ex_hbm, out_hbm,
                 acc, src_v, dst_v, ex_v, r0, r1, r2,
                 g0, g1, g2, s0, s1, s2):
    cid = lax.axis_index("c")
    sid = lax.axis_index("s")
    wid = sid * NC + cid
    rows = [r0, r1, r2]
    gsem = [g0, g1, g2]
    ssem = [s0, s1, s2]

    zeros16 = jnp.zeros((16,), jnp.float32)

    @pl.loop(0, CB)
    def _zero_r0(i):
        for c in range(ZW // 16):
            r0[i, pl.ds(c * 16, 16)] = zeros16

    base_row = sid * ROWS_PT
    for r in range(7):
        pltpu.sync_copy(r0, acc.at[pl.ds(base_row + r * CB, CB)])
    pltpu.sync_copy(r0.at[pl.ds(0, ROWS_PT - 7 * CB)],
                    acc.at[pl.ds(base_row + 7 * CB, ROWS_PT - 7 * CB)])
    plsc.subcore_barrier()

    cbase = wid * (EPW // CB)

    @pl.loop(0, NSUP)
    def _super(ci):
        soff = cbase + ci * KPS
        eoff = soff * CB
        pltpu.sync_copy(src_hbm.at[pl.ds(eoff, CA)], src_v)
        pltpu.sync_copy(dst3_hbm.at[pl.ds(soff, KPS)], dst_v)
        pltpu.sync_copy(ex_hbm.at[pl.ds(eoff, CA)], ex_v)

        def gather(k):
            b = k % 3
            return pltpu.async_copy(
                z_hbm.at[src_v.at[pl.ds(k * CB, CB)]], rows[b], gsem[b])

        def scale(k):
            buf = rows[k % 3]

            @pl.loop(0, CB)
            def _scale(i):
                exb = plsc.load_gather(
                    ex_v, [jnp.full((16,), k * CB, jnp.int32) + i])
                for c in range(ZW // 16):
                    buf[i, pl.ds(c * 16, 16)] = buf[i, pl.ds(c * 16, 16)] * exb

        def scatter(k):
            b = k % 3
            return pltpu.async_copy(rows[b], acc.at[dst_v.at[k]],
                                    ssem[b], add=True)

        gcps = {0: gather(0), 1: gather(1)}
        scps = {}
        for k in range(KPS):
            if k + 2 <= KPS - 1:
                if k >= 1:
                    scps[k - 1].wait()
                gcps[k + 2] = gather(k + 2)
            gcps[k].wait()
            scale(k)
            scps[k] = scatter(k)
        for k in (KPS - 3, KPS - 2, KPS - 1):
            scps[k].wait()

    plsc.subcore_barrier()
    pltpu.sync_copy(acc.at[pl.ds(base_row, ROWS_PT)],
                    out_hbm.at[cid, pl.ds(base_row, ROWS_PT)])


_sc_row = functools.partial(
    pl.kernel,
    out_type=jax.ShapeDtypeStruct((NC, N, ZW), jnp.float32),
    mesh=_SC_MESH,
    scratch_types=[
        pltpu.VMEM_SHARED((N, ZW), jnp.float32),
        pltpu.VMEM((CA,), jnp.int32),
        pltpu.VMEM((KPS, CB), jnp.int32),
        pltpu.VMEM((CA,), jnp.float32),
        pltpu.VMEM((CB, ZW), jnp.float32),
        pltpu.VMEM((CB, ZW), jnp.float32),
        pltpu.VMEM((CB, ZW), jnp.float32),
        pltpu.SemaphoreType.DMA,
        pltpu.SemaphoreType.DMA,
        pltpu.SemaphoreType.DMA,
        pltpu.SemaphoreType.DMA,
        pltpu.SemaphoreType.DMA,
        pltpu.SemaphoreType.DMA,
    ],
    compiler_params=_SC_PARAMS,
)(_sc_row_body)


def _gat_sc(z, st, edge_index, src_flat, dst3):
    ex = _sc_scal(st, edge_index)
    return _sc_row(z, src_flat, dst3, ex)


def _a8(a):
    return jnp.zeros((8, D), jnp.float32).at[0].set(a[:D]).at[1].set(a[D:])


@jax.jit
def kernel(x, edge_index, W1, a1, W2, a2, W3, a3):
    src_flat = edge_index[0]
    dst3 = edge_index[1].reshape(E // CB, CB)
    z1, st1 = _mm_first(x, W1, _a8(a1))
    acc1 = _gat_sc(z1, st1, edge_index, src_flat, dst3)
    z2, st2 = _mm_next(acc1, W2, _a8(a2))
    acc2 = _gat_sc(z2, st2, edge_index, src_flat, dst3)
    z3, st3 = _mm_next(acc2, W3, _a8(a3))
    acc3 = _gat_sc(z3, st3, edge_index, src_flat, dst3)
    return _final(acc3)
